# baseline (speedup 1.0000x reference)
import jax
import jax.numpy as jnp
from jax import lax
from jax.experimental import pallas as pl
from jax.experimental.pallas import tpu as pltpu
from jax.experimental.pallas import tpu_sc as plsc

FEAT = 1024
PFEAT = FEAT // 2
NCLS = 1000
VPAD = 1024
BATCH = 16384

NC, NS = 2, 16
NW = NC * NS
BPW = BATCH // NW
CHUNK = 16
NCHUNK = BPW // CHUNK
NBUF = 4


def _transpose_body(p_ref, o_ref):
    o_ref[0:NCLS, :] = p_ref[...].T


def _transpose(proto):
    return pl.pallas_call(
        _transpose_body,
        out_shape=jax.ShapeDtypeStruct((VPAD, FEAT), jnp.float32),
    )(proto)


def _pack_table(tableT):
    t = tableT.astype(jnp.bfloat16).reshape(VPAD, FEAT // 32, 2, 16)
    t = t.transpose(0, 1, 3, 2).reshape(VPAD, PFEAT, 2)
    return jax.lax.bitcast_convert_type(t, jnp.int32)


def _gather_body(table_hbm, idx_hbm, out_hbm, idx_v, pak_v, rows_v,
                 gsem0, gsem1, gsem2, gsem3, ssem0, ssem1, ssem2, ssem3):
    gsems = (gsem0, gsem1, gsem2, gsem3)
    ssems = (ssem0, ssem1, ssem2, ssem3)
    wid = lax.axis_index("s") * NC + lax.axis_index("c")
    base = wid * BPW
    pltpu.sync_copy(idx_hbm.at[pl.ds(base, BPW)], idx_v)

    def gather(g, b):
        return pltpu.make_async_copy(
            table_hbm.at[idx_v.at[pl.ds(g * CHUNK, CHUNK)]],
            pak_v.at[b],
            gsems[b],
        )

    def scatter(g, b):
        return pltpu.make_async_copy(
            rows_v.at[b],
            out_hbm.at[pl.ds(base + g * CHUNK, CHUNK)],
            ssems[b],
        )

    def widen(b):
        nblk = PFEAT // 16

        @plsc.parallel_loop(0, CHUNK * nblk, unroll=16)
        def _(t):
            r = t // nblk
            k = t % nblk
            v = pak_v[b, r, pl.ds(k * 16, 16)]
            lo = jax.lax.bitcast_convert_type(v << 16, jnp.float32)
            hi = jax.lax.bitcast_convert_type(
                v & jnp.int32(-65536), jnp.float32)
            rows_v[b, r, pl.ds(k * 32, 16)] = lo
            rows_v[b, r, pl.ds(k * 32 + 16, 16)] = hi

    for b in range(NBUF):
        gather(b, b).start()

    def body(j, _):
        for b in range(NBUF):
            g = NBUF * j + b
            gather(g, b).wait()

            @pl.when(g >= NBUF)
            def _():
                scatter(g - NBUF, b).wait()

            widen(b)
            scatter(g, b).start()

            @pl.when(g + NBUF < NCHUNK)
            def _():
                gather(g + NBUF, b).start()
        return 0

    lax.fori_loop(0, NCHUNK // NBUF, body, 0)
    for b in range(NBUF):
        scatter(NCHUNK - NBUF + b, b).wait()


def _sc_gather(table_pak, label):
    mesh = plsc.VectorSubcoreMesh(core_axis_name="c", subcore_axis_name="s")
    return pl.kernel(
        _gather_body,
        out_type=jax.ShapeDtypeStruct((BATCH, FEAT), jnp.float32),
        mesh=mesh,
        scratch_types=[
            pltpu.VMEM((BPW,), jnp.int32),
            pltpu.VMEM((NBUF, CHUNK, PFEAT), jnp.int32),
            pltpu.VMEM((NBUF, CHUNK, FEAT), jnp.float32),
        ] + [pltpu.SemaphoreType.DMA] * 8,
    )(table_pak, label)


def kernel(label, proto):
    table_pak = _pack_table(_transpose(proto))
    return _sc_gather(table_pak, label.astype(jnp.int32))


# speedup vs baseline: 1.0287x; 1.0287x over previous
import jax
import jax.numpy as jnp
from jax import lax
from jax.experimental import pallas as pl
from jax.experimental.pallas import tpu as pltpu
from jax.experimental.pallas import tpu_sc as plsc

FEAT = 1024
NCLS = 1000
VPAD = 1024
BATCH = 16384

NC, NS = 2, 16
NW = NC * NS
BPW = BATCH // NW
CHUNK = 16
NCHUNK = BPW // CHUNK
NBUF = 4


def _transpose_body(p_ref, o_ref):
    o_ref[0:NCLS, :] = p_ref[...].T


def _transpose(proto):
    return pl.pallas_call(
        _transpose_body,
        out_shape=jax.ShapeDtypeStruct((VPAD, FEAT), jnp.float32),
    )(proto)


def _gather_body(table_hbm, idx_hbm, out_hbm, idx_v, rows_v,
                 gsem0, gsem1, gsem2, gsem3, ssem0, ssem1, ssem2, ssem3):
    gsems = (gsem0, gsem1, gsem2, gsem3)
    ssems = (ssem0, ssem1, ssem2, ssem3)
    wid = lax.axis_index("s") * NC + lax.axis_index("c")
    base = wid * BPW
    pltpu.sync_copy(idx_hbm.at[pl.ds(base, BPW)], idx_v)

    def gather(g, b):
        return pltpu.make_async_copy(
            table_hbm.at[idx_v.at[pl.ds(g * CHUNK, CHUNK)]],
            rows_v.at[b],
            gsems[b],
        )

    def scatter(g, b):
        return pltpu.make_async_copy(
            rows_v.at[b],
            out_hbm.at[pl.ds(base + g * CHUNK, CHUNK)],
            ssems[b],
        )

    for b in range(NBUF - 1):
        gather(b, b).start()

    def body(j, _):
        for b in range(NBUF):
            g = NBUF * j + b

            @pl.when(g >= 1)
            def _():
                scatter(g - 1, (b - 1) % NBUF).wait()

            @pl.when(g + NBUF - 1 < NCHUNK)
            def _():
                gather(g + NBUF - 1, (b + NBUF - 1) % NBUF).start()

            gather(g, b).wait()
            scatter(g, b).start()
        return 0

    lax.fori_loop(0, NCHUNK // NBUF, body, 0)
    scatter(NCHUNK - 1, (NCHUNK - 1) % NBUF).wait()


def _sc_gather(tableT, label):
    mesh = plsc.VectorSubcoreMesh(core_axis_name="c", subcore_axis_name="s")
    return pl.kernel(
        _gather_body,
        out_type=jax.ShapeDtypeStruct((BATCH, FEAT), jnp.float32),
        mesh=mesh,
        scratch_types=[
            pltpu.VMEM((BPW,), jnp.int32),
            pltpu.VMEM((NBUF, CHUNK, FEAT), jnp.float32),
        ] + [pltpu.SemaphoreType.DMA] * 8,
    )(tableT, label)


def kernel(label, proto):
    tableT = _transpose(proto)
    return _sc_gather(tableT, label.astype(jnp.int32))
